# Initial kernel scaffold; baseline (speedup 1.0000x reference)
import jax
import jax.numpy as jnp
from jax.experimental import pallas as pl


def kernel(x, W1, b1, W2, b2):
    raise NotImplementedError("write your pallas kernel here")


# speedup vs baseline: 1.6202x; 1.6202x over previous
import jax
import jax.numpy as jnp
from jax.experimental import pallas as pl

_S = 2
_N_PATCH = 64


def _fused_kernel(x_ref, w_ref, b_ref, o_ref, *, L, Lp, E, KP):
    xr = x_ref[0]
    cols = [xr]
    for k in range(1, KP):
        cols.append(jnp.concatenate([xr[k:], xr[:k]], axis=0))
    X = jnp.concatenate(cols, axis=1)
    Y = jnp.dot(X, w_ref[...], preferred_element_type=jnp.float32)
    Y = Y + b_ref[...]
    z = Y[:, :E] * jax.nn.sigmoid(Y[:, E:])
    l_idx = jax.lax.broadcasted_iota(jnp.int32, (Lp, 1), 0)
    z = jnp.where(l_idx < L, z, -jnp.inf)
    o_ref[0] = z.reshape(_N_PATCH, Lp // _N_PATCH, E).max(axis=1)


def kernel(x, W1, b1, W2, b2):
    B, T, C = x.shape
    E, _, K = W1.shape
    L = (T - K) // _S + 1
    Lp = T // _S
    KP = K // _S

    xr = x.reshape(B, Lp, _S * C)

    def fold_w(W):
        return jnp.transpose(W, (2, 1, 0)).reshape(KP, _S * C, E).reshape(KP * _S * C, E)

    Wc = jnp.concatenate([fold_w(W1), fold_w(W2)], axis=1)
    bc = jnp.concatenate([b1, b2]).reshape(1, 2 * E)

    out = pl.pallas_call(
        lambda xref, wref, bref, oref: _fused_kernel(
            xref, wref, bref, oref, L=L, Lp=Lp, E=E, KP=KP),
        grid=(B,),
        in_specs=[
            pl.BlockSpec((1, Lp, _S * C), lambda b: (b, 0, 0)),
            pl.BlockSpec((KP * _S * C, 2 * E), lambda b: (0, 0)),
            pl.BlockSpec((1, 2 * E), lambda b: (0, 0)),
        ],
        out_specs=pl.BlockSpec((1, _N_PATCH, E), lambda b: (b, 0, 0)),
        out_shape=jax.ShapeDtypeStruct((B, _N_PATCH, E), jnp.float32),
    )(xr, Wc, bc)
    return out


# speedup vs baseline: 1.7217x; 1.0627x over previous
import jax
import jax.numpy as jnp
from jax.experimental import pallas as pl

_S = 2
_N_PATCH = 64


def _fused_kernel(x_ref, w_ref, b_ref, o_ref, *, L, Lp, E, KP, BPB):
    xr = x_ref[...].reshape(BPB * Lp, x_ref.shape[2])
    cols = [xr]
    for k in range(1, KP):
        cols.append(jnp.concatenate([xr[k:], xr[:k]], axis=0))
    X = jnp.concatenate(cols, axis=1)
    Y = jnp.dot(X, w_ref[...], preferred_element_type=jnp.float32)
    Y = Y + b_ref[...]
    z = Y[:, :E] * jax.nn.sigmoid(Y[:, E:])
    r_idx = jax.lax.broadcasted_iota(jnp.int32, (BPB * Lp, 1), 0)
    z = jnp.where(r_idx % Lp < L, z, -jnp.inf)
    o_ref[...] = z.reshape(BPB, _N_PATCH, Lp // _N_PATCH, E).max(axis=2)


def kernel(x, W1, b1, W2, b2):
    B, T, C = x.shape
    E, _, K = W1.shape
    L = (T - K) // _S + 1
    Lp = T // _S
    KP = K // _S

    xr = x.reshape(B, Lp, _S * C)

    def fold_w(W):
        return jnp.transpose(W, (2, 1, 0)).reshape(KP, _S * C, E).reshape(KP * _S * C, E)

    Wc = jnp.concatenate([fold_w(W1), fold_w(W2)], axis=1)
    bc = jnp.concatenate([b1, b2]).reshape(1, 2 * E)

    BPB = 4
    out = pl.pallas_call(
        lambda xref, wref, bref, oref: _fused_kernel(
            xref, wref, bref, oref, L=L, Lp=Lp, E=E, KP=KP, BPB=BPB),
        grid=(B // BPB,),
        in_specs=[
            pl.BlockSpec((BPB, Lp, _S * C), lambda b: (b, 0, 0)),
            pl.BlockSpec((KP * _S * C, 2 * E), lambda b: (0, 0)),
            pl.BlockSpec((1, 2 * E), lambda b: (0, 0)),
        ],
        out_specs=pl.BlockSpec((BPB, _N_PATCH, E), lambda b: (b, 0, 0)),
        out_shape=jax.ShapeDtypeStruct((B, _N_PATCH, E), jnp.float32),
    )(xr, Wc, bc)
    return out
